# baseline (speedup 1.0000x reference)
import jax
import jax.numpy as jnp
from jax import lax
from jax.experimental import pallas as pl
from jax.experimental.pallas import tpu as pltpu

_N = 128
_D = 6
_ND = 7
_BF = 13
_AF = 128
_H = 1024
_G = 8
_EXT = 16


def _mol_kernel(atoms_ref, bonds_ref, edges_ref,
                w1_ref, w2_ref, wo_ref,
                out_ref):
    f32 = jnp.float32
    bf16 = jnp.bfloat16

    si = lax.broadcasted_iota(jnp.int32, (_D * _BF, _BF), 0)
    sj = lax.broadcasted_iota(jnp.int32, (_D * _BF, _BF), 1)
    sel = (si % _BF == sj).astype(f32)
    colids = lax.broadcasted_iota(jnp.int32, (_N, _N), 1)
    rowids = lax.broadcasted_iota(jnp.int32, (_N, _N), 0)
    eye_b = (colids == rowids).astype(bf16)

    def run_mol(m):
        x = atoms_ref[m]
        b78 = bonds_ref[m]
        e = edges_ref[m]

        sb = jnp.dot(b78, sel, preferred_element_type=f32)
        sb_ext = jnp.concatenate(
            [sb.astype(bf16),
             jnp.ones((_N, 1), bf16),
             jnp.zeros((_N, _EXT - _BF - 1), bf16)], axis=1)

        onehots = [(e[:, d:d + 1] == colids).astype(bf16) for d in range(_D)]
        A = eye_b
        for oh in onehots:
            A = A + oh
        vbias = [jnp.where(e[:, d:d + 1] >= 0, 0.0, -1e30).astype(bf16)
                 for d in range(_D)]

        deg = jnp.sum((e != -1).astype(f32), axis=1, keepdims=True)
        degb = (deg * jnp.ones((1, 128), f32)).astype(bf16)

        def conv(xin, wcat):
            s_atoms = jnp.dot(A, xin,
                              preferred_element_type=f32)
            lhs = jnp.concatenate([s_atoms.astype(bf16), sb_ext], axis=1)
            z_all = jnp.dot(lhs, wcat,
                            preferred_element_type=f32)
            zsel = z_all[:, 0:128]
            for d in range(1, _ND):
                zsel = jnp.where(degb == d,
                                 z_all[:, d * 128:(d + 1) * 128], zsel)
            return jnp.maximum(zsel, 0.0).astype(bf16)

        def pool(h):
            g = h
            for d in range(_D):
                gd = jnp.dot(onehots[d], h, preferred_element_type=f32)
                g = jnp.maximum(g, gd.astype(bf16) + vbias[d])
            return g

        h1 = conv(x.astype(bf16), w1_ref[...])
        p1 = pool(h1)
        h2 = conv(p1, w2_ref[...])
        p2 = pool(h2)

        lhs = jnp.concatenate([p2, sb_ext], axis=1)
        z = jnp.dot(lhs, wo_ref[...], preferred_element_type=f32)
        mrow = jnp.swapaxes((deg != 0).astype(f32), 0, 1)
        return jnp.dot(mrow, jnp.tanh(z), preferred_element_type=f32)

    for m in range(_G):
        out_ref[m] = run_mol(m)


def kernel(atoms, bonds, edges, W1, b1, W2, b2, Wo, bo):
    B = atoms.shape[0]
    b78 = bonds.reshape(B, _N, _D * _BF)

    def fuse_w(W, b, nout):
        wa = jnp.transpose(W[:, :_AF, :], (1, 0, 2)).reshape(_AF, nout)
        wb = jnp.transpose(W[:, _AF:, :], (1, 0, 2)).reshape(_BF, nout)
        return jnp.concatenate(
            [wa, wb, b.reshape(1, nout),
             jnp.zeros((_EXT - _BF - 1, nout), W.dtype)],
            axis=0).astype(jnp.bfloat16)

    w1c = fuse_w(W1, b1, _ND * 128)
    w2c = fuse_w(W2, b2, _ND * 128)
    woc = jnp.concatenate(
        [Wo, bo.reshape(1, _H), jnp.zeros((_EXT - _BF - 1, _H), Wo.dtype)],
        axis=0).astype(jnp.bfloat16)

    const = lambda i: (0, 0)
    return pl.pallas_call(
        _mol_kernel,
        grid=(B // _G,),
        in_specs=[
            pl.BlockSpec((_G, _N, _AF), lambda i: (i, 0, 0)),
            pl.BlockSpec((_G, _N, _D * _BF), lambda i: (i, 0, 0)),
            pl.BlockSpec((_G, _N, _D), lambda i: (i, 0, 0)),
            pl.BlockSpec((_AF + _EXT, _ND * 128), const),
            pl.BlockSpec((_AF + _EXT, _ND * 128), const),
            pl.BlockSpec((_AF + _EXT, _H), const),
        ],
        out_specs=pl.BlockSpec((_G, 1, _H), lambda i: (i, 0, 0)),
        out_shape=jax.ShapeDtypeStruct((B, 1, _H), jnp.float32),
        compiler_params=pltpu.CompilerParams(
            dimension_semantics=("parallel",)),
    )(atoms, b78, edges, w1c, w2c, woc).reshape(B, _H)


# speedup vs baseline: 1.0118x; 1.0118x over previous
import jax
import jax.numpy as jnp
from jax import lax
from jax.experimental import pallas as pl
from jax.experimental.pallas import tpu as pltpu

_N = 128
_D = 6
_ND = 7
_BF = 13
_AF = 128
_H = 1024
_G = 8
_EXT = 128


def _mol_kernel(atoms_ref, bonds_ref, edges_ref,
                w1_ref, w2_ref, wo_ref,
                out_ref):
    f32 = jnp.float32
    bf16 = jnp.bfloat16

    si = lax.broadcasted_iota(jnp.int32, (_D * _BF, _BF), 0)
    sj = lax.broadcasted_iota(jnp.int32, (_D * _BF, _BF), 1)
    sel = (si % _BF == sj).astype(f32)
    colids = lax.broadcasted_iota(jnp.int32, (_N, _N), 1)
    rowids = lax.broadcasted_iota(jnp.int32, (_N, _N), 0)
    eye_b = (colids == rowids).astype(bf16)

    def run_mol(m):
        x = atoms_ref[m]
        b78 = bonds_ref[m]
        e = edges_ref[m]

        sb = jnp.dot(b78, sel, preferred_element_type=f32)
        sb_ext = jnp.concatenate(
            [sb.astype(bf16),
             jnp.ones((_N, 1), bf16),
             jnp.zeros((_N, _EXT - _BF - 1), bf16)], axis=1)

        onehots = [(e[:, d:d + 1] == colids).astype(bf16) for d in range(_D)]
        A = eye_b
        for oh in onehots:
            A = A + oh
        vbias = [jnp.where(e[:, d:d + 1] >= 0, 0.0, -1e30).astype(f32)
                 for d in range(_D)]

        deg = jnp.sum((e != -1).astype(f32), axis=1, keepdims=True)
        degb = deg * jnp.ones((1, 128), f32)

        def conv(xin, wcat):
            s_atoms = jnp.dot(A, xin,
                              preferred_element_type=f32)
            lhs = jnp.concatenate([s_atoms.astype(bf16), sb_ext], axis=1)
            z_all = jnp.dot(lhs, wcat,
                            preferred_element_type=f32)
            zsel = z_all[:, 0:128]
            for d in range(1, _ND):
                zsel = jnp.where(degb == d,
                                 z_all[:, d * 128:(d + 1) * 128], zsel)
            return jnp.maximum(zsel, 0.0)

        def pool(h):
            g = h
            hb = h.astype(bf16)
            for d in range(_D):
                gd = jnp.dot(onehots[d], hb, preferred_element_type=f32)
                g = jnp.maximum(g, gd + vbias[d])
            return g

        h1 = conv(x.astype(bf16), w1_ref[...])
        p1 = pool(h1)
        h2 = conv(p1.astype(bf16), w2_ref[...])
        p2 = pool(h2)

        lhs = jnp.concatenate([p2.astype(bf16), sb_ext], axis=1)
        z = jnp.dot(lhs, wo_ref[...], preferred_element_type=f32)
        mrow = jnp.swapaxes((deg != 0).astype(f32), 0, 1)
        return jnp.dot(mrow, jnp.tanh(z), preferred_element_type=f32)

    for m in range(_G):
        out_ref[m] = run_mol(m)


def kernel(atoms, bonds, edges, W1, b1, W2, b2, Wo, bo):
    B = atoms.shape[0]
    b78 = bonds.reshape(B, _N, _D * _BF)

    def fuse_w(W, b, nout):
        wa = jnp.transpose(W[:, :_AF, :], (1, 0, 2)).reshape(_AF, nout)
        wb = jnp.transpose(W[:, _AF:, :], (1, 0, 2)).reshape(_BF, nout)
        return jnp.concatenate(
            [wa, wb, b.reshape(1, nout),
             jnp.zeros((_EXT - _BF - 1, nout), W.dtype)],
            axis=0).astype(jnp.bfloat16)

    w1c = fuse_w(W1, b1, _ND * 128)
    w2c = fuse_w(W2, b2, _ND * 128)
    woc = jnp.concatenate(
        [Wo, bo.reshape(1, _H), jnp.zeros((_EXT - _BF - 1, _H), Wo.dtype)],
        axis=0).astype(jnp.bfloat16)

    const = lambda i: (0, 0)
    return pl.pallas_call(
        _mol_kernel,
        grid=(B // _G,),
        in_specs=[
            pl.BlockSpec((_G, _N, _AF), lambda i: (i, 0, 0)),
            pl.BlockSpec((_G, _N, _D * _BF), lambda i: (i, 0, 0)),
            pl.BlockSpec((_G, _N, _D), lambda i: (i, 0, 0)),
            pl.BlockSpec((_AF + _EXT, _ND * 128), const),
            pl.BlockSpec((_AF + _EXT, _ND * 128), const),
            pl.BlockSpec((_AF + _EXT, _H), const),
        ],
        out_specs=pl.BlockSpec((_G, 1, _H), lambda i: (i, 0, 0)),
        out_shape=jax.ShapeDtypeStruct((B, 1, _H), jnp.float32),
        compiler_params=pltpu.CompilerParams(
            dimension_semantics=("parallel",)),
    )(atoms, b78, edges, w1c, w2c, woc).reshape(B, _H)


# speedup vs baseline: 1.0336x; 1.0216x over previous
import jax
import jax.numpy as jnp
from jax import lax
from jax.experimental import pallas as pl
from jax.experimental.pallas import tpu as pltpu

_N = 128
_D = 6
_ND = 7
_BF = 13
_AF = 128
_H = 1024
_G = 16
_EXT = 128


def _mol_kernel(atoms_ref, bonds_ref, edges_ref,
                w1_ref, w2_ref, wo_ref,
                out_ref):
    f32 = jnp.float32
    bf16 = jnp.bfloat16

    si = lax.broadcasted_iota(jnp.int32, (_D * _BF, _BF), 0)
    sj = lax.broadcasted_iota(jnp.int32, (_D * _BF, _BF), 1)
    sel = (si % _BF == sj).astype(f32)
    colids = lax.broadcasted_iota(jnp.int32, (_N, _N), 1)
    rowids = lax.broadcasted_iota(jnp.int32, (_N, _N), 0)
    eye_b = (colids == rowids).astype(bf16)

    def run_mol(m):
        x = atoms_ref[m]
        b78 = bonds_ref[m]
        e = edges_ref[m]

        sb = jnp.dot(b78, sel, preferred_element_type=f32)
        sb_ext = jnp.concatenate(
            [sb.astype(bf16),
             jnp.ones((_N, 1), bf16),
             jnp.zeros((_N, _EXT - _BF - 1), bf16)], axis=1)

        onehots = [(e[:, d:d + 1] == colids).astype(bf16) for d in range(_D)]
        A = eye_b
        for oh in onehots:
            A = A + oh
        vbias = [jnp.where(e[:, d:d + 1] >= 0, 0.0, -1e30).astype(f32)
                 for d in range(_D)]

        deg = jnp.sum((e != -1).astype(f32), axis=1, keepdims=True)
        degb = deg * jnp.ones((1, 128), f32)

        def conv(xin, wcat):
            s_atoms = jnp.dot(A, xin,
                              preferred_element_type=f32)
            lhs = jnp.concatenate([s_atoms.astype(bf16), sb_ext], axis=1)
            z_all = jnp.dot(lhs, wcat,
                            preferred_element_type=f32)
            zsel = z_all[:, 0:128]
            for d in range(1, _ND):
                zsel = jnp.where(degb == d,
                                 z_all[:, d * 128:(d + 1) * 128], zsel)
            return jnp.maximum(zsel, 0.0)

        def pool(h):
            g = h
            hb = h.astype(bf16)
            for d in range(_D):
                gd = jnp.dot(onehots[d], hb, preferred_element_type=f32)
                g = jnp.maximum(g, gd + vbias[d])
            return g

        h1 = conv(x.astype(bf16), w1_ref[...])
        p1 = pool(h1)
        h2 = conv(p1.astype(bf16), w2_ref[...])
        p2 = pool(h2)

        lhs = jnp.concatenate([p2.astype(bf16), sb_ext], axis=1)
        z = jnp.dot(lhs, wo_ref[...], preferred_element_type=f32)
        mrow = jnp.swapaxes((deg != 0).astype(f32), 0, 1)
        return jnp.dot(mrow, jnp.tanh(z), preferred_element_type=f32)

    for m in range(_G):
        out_ref[m] = run_mol(m)


def kernel(atoms, bonds, edges, W1, b1, W2, b2, Wo, bo):
    B = atoms.shape[0]
    b78 = bonds.reshape(B, _N, _D * _BF)

    def fuse_w(W, b, nout):
        wa = jnp.transpose(W[:, :_AF, :], (1, 0, 2)).reshape(_AF, nout)
        wb = jnp.transpose(W[:, _AF:, :], (1, 0, 2)).reshape(_BF, nout)
        return jnp.concatenate(
            [wa, wb, b.reshape(1, nout),
             jnp.zeros((_EXT - _BF - 1, nout), W.dtype)],
            axis=0).astype(jnp.bfloat16)

    w1c = fuse_w(W1, b1, _ND * 128)
    w2c = fuse_w(W2, b2, _ND * 128)
    woc = jnp.concatenate(
        [Wo, bo.reshape(1, _H), jnp.zeros((_EXT - _BF - 1, _H), Wo.dtype)],
        axis=0).astype(jnp.bfloat16)

    const = lambda i: (0, 0)
    return pl.pallas_call(
        _mol_kernel,
        grid=(B // _G,),
        in_specs=[
            pl.BlockSpec((_G, _N, _AF), lambda i: (i, 0, 0)),
            pl.BlockSpec((_G, _N, _D * _BF), lambda i: (i, 0, 0)),
            pl.BlockSpec((_G, _N, _D), lambda i: (i, 0, 0)),
            pl.BlockSpec((_AF + _EXT, _ND * 128), const),
            pl.BlockSpec((_AF + _EXT, _ND * 128), const),
            pl.BlockSpec((_AF + _EXT, _H), const),
        ],
        out_specs=pl.BlockSpec((_G, 1, _H), lambda i: (i, 0, 0)),
        out_shape=jax.ShapeDtypeStruct((B, 1, _H), jnp.float32),
        compiler_params=pltpu.CompilerParams(
            dimension_semantics=("parallel",)),
    )(atoms, b78, edges, w1c, w2c, woc).reshape(B, _H)
